# baseline (speedup 1.0000x reference)
import functools

import jax
import jax.numpy as jnp
from jax.experimental import pallas as pl
from jax.experimental.pallas import tpu as pltpu

_TOPK = 64


def _sae_body(x_ref, enc_ref, dec_ref, bpre_ref, benc_ref, zn_ref, xt_ref,
              *, topk):
    x = x_ref[...]
    xb = x - bpre_ref[...]
    logits = jax.lax.dot_general(
        xb, enc_ref[...], (((1,), (0,)), ((), ())),
        preferred_element_type=jnp.float32,
        precision=jax.lax.Precision.DEFAULT)
    z = jnp.maximum(logits + benc_ref[...], 0.0)

    R = z.shape[0]
    u = jax.lax.bitcast_convert_type(z, jnp.int32)

    def cond(state):
        b, acc, cnt = state
        return (b >= 0) & jnp.any(cnt != topk)

    def body(state):
        b, acc, cnt = state
        cand = acc | (1 << b)
        c = jnp.sum((u >= cand).astype(jnp.int32), axis=1, keepdims=True)
        take = c >= topk
        return (b - 1, jnp.where(take, cand, acc), jnp.where(take, c, cnt))

    _, acc, _ = jax.lax.while_loop(
        cond, body,
        (jnp.int32(30), jnp.zeros((R, 1), jnp.int32),
         jnp.full((R, 1), u.shape[1], jnp.int32)))
    thr = jax.lax.bitcast_convert_type(acc, jnp.float32)

    zs = jnp.where(z >= thr, z, 0.0)
    zn_ref[...] = zs
    xt = jax.lax.dot_general(
        zs, dec_ref[...], (((1,), (0,)), ((), ())),
        preferred_element_type=jnp.float32,
        precision=jax.lax.Precision.DEFAULT)
    xt_ref[...] = xt + bpre_ref[...]


def kernel(zL, dictionary_enc, dictionary_dec, bias_pre, bias_enc):
    B, D, L, H = zL.shape
    M = dictionary_enc.shape[0]
    N = B * D * L
    R = 256 if N % 256 == 0 else N
    grid = N // R

    x = zL.reshape(N, H)
    enc_t = dictionary_enc.T
    dec_t = dictionary_dec.T
    bpre = bias_pre.reshape(1, H)
    benc = bias_enc.reshape(1, M)

    z_n, x_tgt = pl.pallas_call(
        functools.partial(_sae_body, topk=_TOPK),
        grid=(grid,),
        in_specs=[
            pl.BlockSpec((R, H), lambda i: (i, 0)),
            pl.BlockSpec((H, M), lambda i: (0, 0)),
            pl.BlockSpec((M, H), lambda i: (0, 0)),
            pl.BlockSpec((1, H), lambda i: (0, 0)),
            pl.BlockSpec((1, M), lambda i: (0, 0)),
        ],
        out_specs=[
            pl.BlockSpec((R, M), lambda i: (i, 0)),
            pl.BlockSpec((R, H), lambda i: (i, 0)),
        ],
        out_shape=[
            jax.ShapeDtypeStruct((N, M), jnp.float32),
            jax.ShapeDtypeStruct((N, H), jnp.float32),
        ],
    )(x, enc_t, dec_t, bpre, benc)

    return z_n.reshape(B, D, L, M), x_tgt.reshape(B, D, L, H)


# speedup vs baseline: 1.0828x; 1.0828x over previous
import functools

import jax
import jax.numpy as jnp
from jax.experimental import pallas as pl
from jax.experimental.pallas import tpu as pltpu

_TOPK = 64


def _sae_body(x_ref, enc_ref, dec_ref, bpre_ref, benc_ref, zn_ref, xt_ref,
              *, topk):
    x = x_ref[...]
    xb = x - bpre_ref[...]
    logits = jax.lax.dot_general(
        xb, enc_ref[...], (((1,), (0,)), ((), ())),
        preferred_element_type=jnp.float32,
        precision=jax.lax.Precision.DEFAULT)
    z = jnp.maximum(logits + benc_ref[...], 0.0)

    u = jax.lax.bitcast_convert_type(z, jnp.int32)
    acc = jnp.zeros((z.shape[0], 1), jnp.int32)
    for b in range(30, -1, -1):
        cand = acc | (1 << b)
        cnt = jnp.sum((u >= cand).astype(jnp.int32), axis=1, keepdims=True)
        acc = jnp.where(cnt >= topk, cand, acc)
    thr = jax.lax.bitcast_convert_type(acc, jnp.float32)

    zs = jnp.where(z >= thr, z, 0.0)
    zn_ref[...] = zs
    xt = jax.lax.dot_general(
        zs, dec_ref[...], (((1,), (0,)), ((), ())),
        preferred_element_type=jnp.float32,
        precision=jax.lax.Precision.DEFAULT)
    xt_ref[...] = xt + bpre_ref[...]


def kernel(zL, dictionary_enc, dictionary_dec, bias_pre, bias_enc):
    B, D, L, H = zL.shape
    M = dictionary_enc.shape[0]
    N = B * D * L
    R = 512 if N % 512 == 0 else N
    grid = N // R

    x = zL.reshape(N, H)
    enc_t = dictionary_enc.T
    dec_t = dictionary_dec.T
    bpre = bias_pre.reshape(1, H)
    benc = bias_enc.reshape(1, M)

    z_n, x_tgt = pl.pallas_call(
        functools.partial(_sae_body, topk=_TOPK),
        grid=(grid,),
        in_specs=[
            pl.BlockSpec((R, H), lambda i: (i, 0)),
            pl.BlockSpec((H, M), lambda i: (0, 0)),
            pl.BlockSpec((M, H), lambda i: (0, 0)),
            pl.BlockSpec((1, H), lambda i: (0, 0)),
            pl.BlockSpec((1, M), lambda i: (0, 0)),
        ],
        out_specs=[
            pl.BlockSpec((R, M), lambda i: (i, 0)),
            pl.BlockSpec((R, H), lambda i: (i, 0)),
        ],
        out_shape=[
            jax.ShapeDtypeStruct((N, M), jnp.float32),
            jax.ShapeDtypeStruct((N, H), jnp.float32),
        ],
    )(x, enc_t, dec_t, bpre, benc)

    return z_n.reshape(B, D, L, M), x_tgt.reshape(B, D, L, H)


# speedup vs baseline: 1.1551x; 1.0668x over previous
import functools

import jax
import jax.numpy as jnp
from jax.experimental import pallas as pl
from jax.experimental.pallas import tpu as pltpu

_TOPK = 64


def _sae_body(x_ref, enc_ref, dec_ref, bpre_ref, benc_ref, zn_ref, xt_ref,
              *, topk):
    x = x_ref[...]
    xb = x - bpre_ref[...]
    logits = jax.lax.dot_general(
        xb, enc_ref[...], (((1,), (0,)), ((), ())),
        preferred_element_type=jnp.float32,
        precision=jax.lax.Precision.DEFAULT)
    z = jnp.maximum(logits + benc_ref[...], 0.0)

    acc = jnp.zeros((z.shape[0], 1), jnp.int32)
    for b in range(30, -1, -1):
        cand = acc | (1 << b)
        candf = jax.lax.bitcast_convert_type(cand, jnp.float32)
        cnt = jnp.sum(jnp.where(z >= candf, 1.0, 0.0), axis=1, keepdims=True)
        acc = jnp.where(cnt >= topk, cand, acc)
    thr = jax.lax.bitcast_convert_type(acc, jnp.float32)

    zs = jnp.where(z >= thr, z, 0.0)
    zn_ref[...] = zs
    xt = jax.lax.dot_general(
        zs, dec_ref[...], (((1,), (0,)), ((), ())),
        preferred_element_type=jnp.float32,
        precision=jax.lax.Precision.DEFAULT)
    xt_ref[...] = xt + bpre_ref[...]


def kernel(zL, dictionary_enc, dictionary_dec, bias_pre, bias_enc):
    B, D, L, H = zL.shape
    M = dictionary_enc.shape[0]
    N = B * D * L
    R = 512 if N % 512 == 0 else N
    grid = N // R

    x = zL.reshape(N, H)
    enc_t = dictionary_enc.T
    dec_t = dictionary_dec.T
    bpre = bias_pre.reshape(1, H)
    benc = bias_enc.reshape(1, M)

    z_n, x_tgt = pl.pallas_call(
        functools.partial(_sae_body, topk=_TOPK),
        grid=(grid,),
        in_specs=[
            pl.BlockSpec((R, H), lambda i: (i, 0)),
            pl.BlockSpec((H, M), lambda i: (0, 0)),
            pl.BlockSpec((M, H), lambda i: (0, 0)),
            pl.BlockSpec((1, H), lambda i: (0, 0)),
            pl.BlockSpec((1, M), lambda i: (0, 0)),
        ],
        out_specs=[
            pl.BlockSpec((R, M), lambda i: (i, 0)),
            pl.BlockSpec((R, H), lambda i: (i, 0)),
        ],
        out_shape=[
            jax.ShapeDtypeStruct((N, M), jnp.float32),
            jax.ShapeDtypeStruct((N, H), jnp.float32),
        ],
    )(x, enc_t, dec_t, bpre, benc)

    return z_n.reshape(B, D, L, M), x_tgt.reshape(B, D, L, H)


# speedup vs baseline: 1.1929x; 1.0328x over previous
import functools

import jax
import jax.numpy as jnp
from jax.experimental import pallas as pl
from jax.experimental.pallas import tpu as pltpu

_TOPK = 64


def _sae_body(x_ref, enc_ref, dec_ref, bpre_ref, benc_ref, zn_ref, xt_ref,
              *, topk):
    x = x_ref[...]
    xb = x - bpre_ref[...]
    logits = jax.lax.dot_general(
        xb, enc_ref[...], (((1,), (1,)), ((), ())),
        preferred_element_type=jnp.float32,
        precision=jax.lax.Precision.DEFAULT)
    z = jnp.maximum(logits + benc_ref[...], 0.0)

    acc = jnp.zeros((z.shape[0], 1), jnp.int32)
    for b in range(30, -1, -1):
        cand = acc | (1 << b)
        candf = jax.lax.bitcast_convert_type(cand, jnp.float32)
        cnt = jnp.sum(jnp.where(z >= candf, 1.0, 0.0), axis=1, keepdims=True)
        acc = jnp.where(cnt >= topk, cand, acc)
    thr = jax.lax.bitcast_convert_type(acc, jnp.float32)

    zs = jnp.where(z >= thr, z, 0.0)
    zn_ref[...] = zs
    xt = jax.lax.dot_general(
        zs, dec_ref[...], (((1,), (1,)), ((), ())),
        preferred_element_type=jnp.float32,
        precision=jax.lax.Precision.DEFAULT)
    xt_ref[...] = xt + bpre_ref[...]


def kernel(zL, dictionary_enc, dictionary_dec, bias_pre, bias_enc):
    B, D, L, H = zL.shape
    M = dictionary_enc.shape[0]
    N = B * D * L
    R = 512 if N % 512 == 0 else N
    grid = N // R

    x = zL.reshape(N, H)
    enc_t = dictionary_enc
    dec_t = dictionary_dec
    bpre = bias_pre.reshape(1, H)
    benc = bias_enc.reshape(1, M)

    z_n, x_tgt = pl.pallas_call(
        functools.partial(_sae_body, topk=_TOPK),
        grid=(grid,),
        in_specs=[
            pl.BlockSpec((R, H), lambda i: (i, 0)),
            pl.BlockSpec((M, H), lambda i: (0, 0)),
            pl.BlockSpec((H, M), lambda i: (0, 0)),
            pl.BlockSpec((1, H), lambda i: (0, 0)),
            pl.BlockSpec((1, M), lambda i: (0, 0)),
        ],
        out_specs=[
            pl.BlockSpec((R, M), lambda i: (i, 0)),
            pl.BlockSpec((R, H), lambda i: (i, 0)),
        ],
        out_shape=[
            jax.ShapeDtypeStruct((N, M), jnp.float32),
            jax.ShapeDtypeStruct((N, H), jnp.float32),
        ],
    )(x, enc_t, dec_t, bpre, benc)

    return z_n.reshape(B, D, L, M), x_tgt.reshape(B, D, L, H)


# speedup vs baseline: 1.2966x; 1.0869x over previous
import functools

import jax
import jax.numpy as jnp
from jax.experimental import pallas as pl
from jax.experimental.pallas import tpu as pltpu

_TOPK = 64


def _sae_body(x_ref, enc_ref, dec_ref, bpre_ref, benc_ref, zn_ref, xt_ref,
              acc_ref, *, topk):
    x = x_ref[...]
    xb = x - bpre_ref[...]
    logits = jax.lax.dot_general(
        xb, enc_ref[...], (((1,), (1,)), ((), ())),
        preferred_element_type=jnp.float32,
        precision=jax.lax.Precision.DEFAULT)
    z = jnp.maximum(logits + benc_ref[...], 0.0)

    topkf = jnp.float32(topk)
    acc = jnp.zeros((z.shape[0], 1), jnp.int32)
    cnta = jnp.full((z.shape[0], 1), jnp.float32(z.shape[1]))
    for b in range(30, 5, -1):
        cand = acc | (1 << b)
        candf = jax.lax.bitcast_convert_type(cand, jnp.float32)
        cnt = jnp.sum(jnp.where(z >= candf, 1.0, 0.0), axis=1, keepdims=True)
        take = cnt >= topkf
        acc = jnp.where(take, cand, acc)
        cnta = jnp.where(take, cnt, cnta)
    acc_ref[...] = acc

    @pl.when(jnp.any(cnta != topkf))
    def _refine():
        a = acc_ref[...]
        for b in range(5, -1, -1):
            cand = a | (1 << b)
            candf = jax.lax.bitcast_convert_type(cand, jnp.float32)
            cnt = jnp.sum(jnp.where(z >= candf, 1.0, 0.0),
                          axis=1, keepdims=True)
            a = jnp.where(cnt >= topkf, cand, a)
        acc_ref[...] = a

    thr = jax.lax.bitcast_convert_type(acc_ref[...], jnp.float32)

    zs = jnp.where(z >= thr, z, 0.0)
    zn_ref[...] = zs
    xt = jax.lax.dot_general(
        zs, dec_ref[...], (((1,), (1,)), ((), ())),
        preferred_element_type=jnp.float32,
        precision=jax.lax.Precision.DEFAULT)
    xt_ref[...] = xt + bpre_ref[...]


def kernel(zL, dictionary_enc, dictionary_dec, bias_pre, bias_enc):
    B, D, L, H = zL.shape
    M = dictionary_enc.shape[0]
    N = B * D * L
    R = 512 if N % 512 == 0 else N
    grid = N // R

    x = zL.reshape(N, H)
    enc_t = dictionary_enc
    dec_t = dictionary_dec
    bpre = bias_pre.reshape(1, H)
    benc = bias_enc.reshape(1, M)

    z_n, x_tgt = pl.pallas_call(
        functools.partial(_sae_body, topk=_TOPK),
        grid=(grid,),
        in_specs=[
            pl.BlockSpec((R, H), lambda i: (i, 0)),
            pl.BlockSpec((M, H), lambda i: (0, 0)),
            pl.BlockSpec((H, M), lambda i: (0, 0)),
            pl.BlockSpec((1, H), lambda i: (0, 0)),
            pl.BlockSpec((1, M), lambda i: (0, 0)),
        ],
        out_specs=[
            pl.BlockSpec((R, M), lambda i: (i, 0)),
            pl.BlockSpec((R, H), lambda i: (i, 0)),
        ],
        out_shape=[
            jax.ShapeDtypeStruct((N, M), jnp.float32),
            jax.ShapeDtypeStruct((N, H), jnp.float32),
        ],
        scratch_shapes=[pltpu.VMEM((R, 1), jnp.int32)],
    )(x, enc_t, dec_t, bpre, benc)

    return z_n.reshape(B, D, L, M), x_tgt.reshape(B, D, L, H)
